# Initial kernel scaffold; baseline (speedup 1.0000x reference)
import jax
import jax.numpy as jnp
from jax.experimental import pallas as pl


def kernel(edge_index, nodes_feat, edges_feat, nodes_num_norm_sqrt, edges_num_norm_sqrt, W1, b1, gamma1, beta1, W2, b2, gamma2, beta2):
    raise NotImplementedError("write your pallas kernel here")


# speedup vs baseline: 5.9308x; 5.9308x over previous
import jax
import jax.numpy as jnp
from jax import lax
from jax.experimental import pallas as pl
from jax.experimental.pallas import tpu as pltpu
from jax.experimental.pallas import tpu_sc as plsc

N = 10000
D = 128
E = 320000

K = 80
NC = 2
NS = 16
NW = NC * NS
CHUNKS = E // K
CH_DEG = CHUNKS // NS
CH_AGG = CHUNKS // NW
NP = 10240
RPT = NP // NS

_MESH = plsc.VectorSubcoreMesh(core_axis_name="c", subcore_axis_name="s")


def _zero_vmem_2d(ref, rows, width):
    z = jnp.zeros((16,), jnp.float32)

    def body(i, _):
        for k in range(width // 16):
            ref[i, pl.ds(16 * k, 16)] = z
        return 0

    lax.fori_loop(0, rows, body, 0)


def _fill_vmem_2d(ref, rows, width, value):
    v = jnp.full((16,), value, jnp.float32)

    def body(i, _):
        for k in range(width // 16):
            ref[i, pl.ds(16 * k, 16)] = v
        return 0

    lax.fori_loop(0, rows, body, 0)


DEG_EPS = 2.0 ** -12


def _deg_body(e_hbm, out_hbm, shared_deg, idx_v, val_v):
    c = lax.axis_index("c")
    s = lax.axis_index("s")
    w = c * NS + s
    r0 = s * RPT

    _zero_vmem_2d(val_v, K, D)
    for q in range(RPT // K):
        pltpu.sync_copy(val_v, shared_deg.at[pl.ds(r0 + K * q, K)])
    plsc.subcore_barrier()

    pltpu.sync_copy(e_hbm.at[0, w], idx_v)
    _fill_vmem_2d(val_v, K, D, 1.0)

    def body(j, _):
        pltpu.sync_copy(val_v, shared_deg.at[idx_v.at[j]], add=True)
        return 0

    lax.fori_loop(0, CH_AGG, body, 0)

    pltpu.sync_copy(e_hbm.at[1, w], idx_v)
    _fill_vmem_2d(val_v, K, D, DEG_EPS)
    lax.fori_loop(0, CH_AGG, body, 0)
    plsc.subcore_barrier()

    pltpu.sync_copy(shared_deg.at[pl.ds(r0, RPT)], out_hbm.at[c, s])


def _deg_call(e_agg):
    return pl.kernel(
        _deg_body,
        out_type=jax.ShapeDtypeStruct((NC, NS, RPT, D), jnp.float32),
        mesh=_MESH,
        scratch_types=[
            pltpu.VMEM_SHARED((NP, D), jnp.float32),
            pltpu.VMEM((CH_AGG, K), jnp.int32),
            pltpu.VMEM((K, D), jnp.float32),
        ],
    )(e_agg)


def _agg_body(x_hbm, e_hbm, out_hbm, shared_agg, idx_s, idx_d, rows_v, sem):
    c = lax.axis_index("c")
    s = lax.axis_index("s")
    w = c * NS + s
    r0 = s * RPT

    pltpu.sync_copy(e_hbm.at[0, w], idx_s)
    pltpu.sync_copy(e_hbm.at[1, w], idx_d)

    _zero_vmem_2d(rows_v, K, D)
    for q in range(RPT // K):
        pltpu.sync_copy(rows_v, shared_agg.at[pl.ds(r0 + K * q, K)])
    plsc.subcore_barrier()

    def body(j, _):
        pltpu.async_copy(x_hbm.at[idx_s.at[j]], rows_v, sem).wait()
        pltpu.sync_copy(rows_v, shared_agg.at[idx_d.at[j]], add=True)
        return 0

    lax.fori_loop(0, CH_AGG, body, 0)
    plsc.subcore_barrier()

    pltpu.sync_copy(shared_agg.at[pl.ds(r0, RPT)], out_hbm.at[c, s])


def _agg_call(x, e_agg):
    return pl.kernel(
        _agg_body,
        out_type=jax.ShapeDtypeStruct((NC, NS, RPT, D), jnp.float32),
        mesh=_MESH,
        scratch_types=[
            pltpu.VMEM_SHARED((NP, D), jnp.float32),
            pltpu.VMEM((CH_AGG, K), jnp.int32),
            pltpu.VMEM((CH_AGG, K), jnp.int32),
            pltpu.VMEM((K, D), jnp.float32),
            pltpu.SemaphoreType.DMA,
        ],
    )(x, e_agg)


def _pre_body(nf_ref, dp_ref, x1_ref, ds_ref, dd_ref):
    t = dp_ref[0] + dp_ref[1]
    dsrc = jnp.floor(t)
    ddst = (t - dsrc) * (1.0 / DEG_EPS)
    dinv_s = lax.rsqrt(jnp.maximum(dsrc, 1.0))
    dinv_d = lax.rsqrt(jnp.maximum(ddst, 1.0))
    ds_ref[...] = dinv_s
    dd_ref[...] = dinv_d
    x1_ref[...] = nf_ref[...] * dinv_s


def _pre_call(nodes_feat, deg_partials):
    return pl.pallas_call(
        _pre_body,
        out_shape=(
            jax.ShapeDtypeStruct((N, D), jnp.float32),
            jax.ShapeDtypeStruct((N, 1), jnp.float32),
            jax.ShapeDtypeStruct((N, 1), jnp.float32),
        ),
    )(nodes_feat, deg_partials)


def _dense_common(p_ref, dd_ref, snorm_ref, w_ref, b_ref, g_ref, be_ref,
                  h_ref):
    agg = (p_ref[0] + p_ref[1]) * dd_ref[...]
    y = jnp.dot(agg, w_ref[...], preferred_element_type=jnp.float32)
    y = (y + b_ref[...]) * snorm_ref[...]
    mu = jnp.mean(y, axis=0, keepdims=True)
    yc = y - mu
    var = jnp.mean(yc * yc, axis=0, keepdims=True)
    yn = yc * lax.rsqrt(var + 1e-5) * g_ref[...] + be_ref[...]
    return h_ref[...] + jnp.maximum(yn, 0.0)


def _layer1_body(p_ref, dd_ref, snorm_ref, w_ref, b_ref, g_ref, be_ref,
                 h_ref, ds_ref, h1_ref, x2_ref):
    h1 = _dense_common(p_ref, dd_ref, snorm_ref, w_ref, b_ref, g_ref,
                       be_ref, h_ref)
    h1_ref[...] = h1
    x2_ref[...] = h1 * ds_ref[...]


def _layer1_call(p, dinv_dst, snorm, w, b, g, be, h_in, dinv_src):
    return pl.pallas_call(
        _layer1_body,
        out_shape=(
            jax.ShapeDtypeStruct((N, D), jnp.float32),
            jax.ShapeDtypeStruct((N, D), jnp.float32),
        ),
    )(p, dinv_dst, snorm, w, b, g, be, h_in, dinv_src)


def _layer2_body(p_ref, dd_ref, snorm_ref, w_ref, b_ref, g_ref, be_ref,
                 h_ref, hg_ref):
    h2 = _dense_common(p_ref, dd_ref, snorm_ref, w_ref, b_ref, g_ref,
                       be_ref, h_ref)
    hg_ref[...] = jnp.mean(h2, axis=0, keepdims=True)


def _layer2_call(p, dinv_dst, snorm, w, b, g, be, h_in):
    return pl.pallas_call(
        _layer2_body,
        out_shape=jax.ShapeDtypeStruct((1, D), jnp.float32),
    )(p, dinv_dst, snorm, w, b, g, be, h_in)


def kernel(edge_index, nodes_feat, edges_feat, nodes_num_norm_sqrt,
           edges_num_norm_sqrt, W1, b1, gamma1, beta1, W2, b2, gamma2, beta2):
    del edges_feat, edges_num_norm_sqrt
    ei = edge_index.astype(jnp.int32)
    e_agg = ei.reshape(2, NW, CH_AGG, K)

    deg_p = _deg_call(e_agg).reshape(NC, NP, D)[:, :N, 0:1]

    b1r, g1r, be1r = b1.reshape(1, D), gamma1.reshape(1, D), beta1.reshape(1, D)
    b2r, g2r, be2r = b2.reshape(1, D), gamma2.reshape(1, D), beta2.reshape(1, D)
    snorm = nodes_num_norm_sqrt

    x1, dinv_s, dinv_d = _pre_call(nodes_feat, deg_p)
    p1 = _agg_call(x1, e_agg).reshape(2, NP, D)[:, :N]
    h1, x2 = _layer1_call(p1, dinv_d, snorm, W1, b1r, g1r, be1r,
                          nodes_feat, dinv_s)
    p2 = _agg_call(x2, e_agg).reshape(2, NP, D)[:, :N]
    hg = _layer2_call(p2, dinv_d, snorm, W2, b2r, g2r, be2r, h1)
    return hg


# speedup vs baseline: 6.5023x; 1.0964x over previous
import jax
import jax.numpy as jnp
from jax import lax
from jax.experimental import pallas as pl
from jax.experimental.pallas import tpu as pltpu
from jax.experimental.pallas import tpu_sc as plsc

N = 10000
D = 128
E = 320000

K = 128
NC = 2
NS = 16
NW = NC * NS
CH_AGG = 80
E_PAD = NW * CH_AGG * K
NP = 10240
RPT = NP // NS

_MESH = plsc.VectorSubcoreMesh(core_axis_name="c", subcore_axis_name="s")


def _zero_vmem_2d(ref, rows, width):
    z = jnp.zeros((16,), jnp.float32)

    def body(i, _):
        for k in range(width // 16):
            ref[i, pl.ds(16 * k, 16)] = z
        return 0

    lax.fori_loop(0, rows, body, 0)


def _fill_vmem_2d(ref, rows, width, value):
    v = jnp.full((16,), value, jnp.float32)

    def body(i, _):
        for k in range(width // 16):
            ref[i, pl.ds(16 * k, 16)] = v
        return 0

    lax.fori_loop(0, rows, body, 0)


DEG_EPS = 2.0 ** -12


def _deg_body(e_hbm, out_hbm, shared_deg, idx_v, val_v):
    c = lax.axis_index("c")
    s = lax.axis_index("s")
    w = c * NS + s
    r0 = s * RPT

    _zero_vmem_2d(val_v, K, D)
    for q in range(RPT // K):
        pltpu.sync_copy(val_v, shared_deg.at[pl.ds(r0 + K * q, K)])
    plsc.subcore_barrier()

    pltpu.sync_copy(e_hbm.at[0, w], idx_v)
    _fill_vmem_2d(val_v, K, D, 1.0)

    def body(j, _):
        pltpu.sync_copy(val_v, shared_deg.at[idx_v.at[j]], add=True)
        return 0

    lax.fori_loop(0, CH_AGG, body, 0)

    pltpu.sync_copy(e_hbm.at[1, w], idx_v)
    _fill_vmem_2d(val_v, K, D, DEG_EPS)
    lax.fori_loop(0, CH_AGG, body, 0)
    plsc.subcore_barrier()

    pltpu.sync_copy(shared_deg.at[pl.ds(r0, RPT)], out_hbm.at[c, s])


def _deg_call(e_agg):
    return pl.kernel(
        _deg_body,
        out_type=jax.ShapeDtypeStruct((NC, NS, RPT, D), jnp.float32),
        mesh=_MESH,
        scratch_types=[
            pltpu.VMEM_SHARED((NP, D), jnp.float32),
            pltpu.VMEM((CH_AGG, K), jnp.int32),
            pltpu.VMEM((K, D), jnp.float32),
        ],
    )(e_agg)


def _agg_body(x_hbm, e_hbm, out_hbm, shared_agg, idx_s, idx_d, rows_v, sem):
    c = lax.axis_index("c")
    s = lax.axis_index("s")
    w = c * NS + s
    r0 = s * RPT

    pltpu.sync_copy(e_hbm.at[0, w], idx_s)
    pltpu.sync_copy(e_hbm.at[1, w], idx_d)

    _zero_vmem_2d(rows_v, K, D)
    for q in range(RPT // K):
        pltpu.sync_copy(rows_v, shared_agg.at[pl.ds(r0 + K * q, K)])
    plsc.subcore_barrier()

    def body(j, _):
        pltpu.async_copy(x_hbm.at[idx_s.at[j]], rows_v, sem).wait()
        pltpu.sync_copy(rows_v, shared_agg.at[idx_d.at[j]], add=True)
        return 0

    lax.fori_loop(0, CH_AGG, body, 0)
    plsc.subcore_barrier()

    pltpu.sync_copy(shared_agg.at[pl.ds(r0, RPT)], out_hbm.at[c, s])


def _agg_call(x, e_agg):
    return pl.kernel(
        _agg_body,
        out_type=jax.ShapeDtypeStruct((NC, NS, RPT, D), jnp.float32),
        mesh=_MESH,
        scratch_types=[
            pltpu.VMEM_SHARED((NP, D), jnp.float32),
            pltpu.VMEM((CH_AGG, K), jnp.int32),
            pltpu.VMEM((CH_AGG, K), jnp.int32),
            pltpu.VMEM((K, D), jnp.float32),
            pltpu.SemaphoreType.DMA,
        ],
    )(x, e_agg)


def _pre_body(nf_ref, dp_ref, x1_ref, ds_ref, dd_ref):
    t = dp_ref[0] + dp_ref[1]
    dsrc = jnp.floor(t)
    ddst = (t - dsrc) * (1.0 / DEG_EPS)
    dinv_s = lax.rsqrt(jnp.maximum(dsrc, 1.0))
    dinv_d = lax.rsqrt(jnp.maximum(ddst, 1.0))
    ds_ref[...] = dinv_s
    dd_ref[...] = dinv_d
    x1_ref[pl.ds(0, N), :] = nf_ref[...] * dinv_s
    x1_ref[pl.ds(N, NP - N), :] = jnp.zeros((NP - N, D), jnp.float32)


def _pre_call(nodes_feat, deg_partials):
    return pl.pallas_call(
        _pre_body,
        out_shape=(
            jax.ShapeDtypeStruct((NP, D), jnp.float32),
            jax.ShapeDtypeStruct((N, 1), jnp.float32),
            jax.ShapeDtypeStruct((N, 1), jnp.float32),
        ),
    )(nodes_feat, deg_partials)


def _dense_common(p_ref, dd_ref, snorm_ref, w_ref, b_ref, g_ref, be_ref,
                  h_ref):
    agg = (p_ref[0] + p_ref[1]) * dd_ref[...]
    y = jnp.dot(agg, w_ref[...], preferred_element_type=jnp.float32)
    y = (y + b_ref[...]) * snorm_ref[...]
    mu = jnp.mean(y, axis=0, keepdims=True)
    yc = y - mu
    var = jnp.mean(yc * yc, axis=0, keepdims=True)
    yn = yc * lax.rsqrt(var + 1e-5) * g_ref[...] + be_ref[...]
    return h_ref[...] + jnp.maximum(yn, 0.0)


def _layer1_body(p_ref, dd_ref, snorm_ref, w_ref, b_ref, g_ref, be_ref,
                 h_ref, ds_ref, h1_ref, x2_ref):
    h1 = _dense_common(p_ref, dd_ref, snorm_ref, w_ref, b_ref, g_ref,
                       be_ref, h_ref)
    h1_ref[...] = h1
    x2_ref[pl.ds(0, N), :] = h1 * ds_ref[...]
    x2_ref[pl.ds(N, NP - N), :] = jnp.zeros((NP - N, D), jnp.float32)


def _layer1_call(p, dinv_dst, snorm, w, b, g, be, h_in, dinv_src):
    return pl.pallas_call(
        _layer1_body,
        out_shape=(
            jax.ShapeDtypeStruct((N, D), jnp.float32),
            jax.ShapeDtypeStruct((NP, D), jnp.float32),
        ),
    )(p, dinv_dst, snorm, w, b, g, be, h_in, dinv_src)


def _layer2_body(p_ref, dd_ref, snorm_ref, w_ref, b_ref, g_ref, be_ref,
                 h_ref, hg_ref):
    h2 = _dense_common(p_ref, dd_ref, snorm_ref, w_ref, b_ref, g_ref,
                       be_ref, h_ref)
    hg_ref[...] = jnp.mean(h2, axis=0, keepdims=True)


def _layer2_call(p, dinv_dst, snorm, w, b, g, be, h_in):
    return pl.pallas_call(
        _layer2_body,
        out_shape=jax.ShapeDtypeStruct((1, D), jnp.float32),
    )(p, dinv_dst, snorm, w, b, g, be, h_in)


def kernel(edge_index, nodes_feat, edges_feat, nodes_num_norm_sqrt,
           edges_num_norm_sqrt, W1, b1, gamma1, beta1, W2, b2, gamma2, beta2):
    del edges_feat, edges_num_norm_sqrt
    ei = edge_index.astype(jnp.int32)
    padv = N + (jnp.arange(E_PAD - E, dtype=jnp.int32) % (NP - N))
    e_all = jnp.concatenate([ei, jnp.broadcast_to(padv, (2, E_PAD - E))], axis=1)
    e_agg = e_all.reshape(2, NW, CH_AGG, K)

    deg_p = _deg_call(e_agg).reshape(NC, NP, D)[:, :N, 0:1]

    b1r, g1r, be1r = b1.reshape(1, D), gamma1.reshape(1, D), beta1.reshape(1, D)
    b2r, g2r, be2r = b2.reshape(1, D), gamma2.reshape(1, D), beta2.reshape(1, D)
    snorm = nodes_num_norm_sqrt

    x1, dinv_s, dinv_d = _pre_call(nodes_feat, deg_p)
    p1 = _agg_call(x1, e_agg).reshape(2, NP, D)[:, :N]
    h1, x2 = _layer1_call(p1, dinv_d, snorm, W1, b1r, g1r, be1r,
                          nodes_feat, dinv_s)
    p2 = _agg_call(x2, e_agg).reshape(2, NP, D)[:, :N]
    hg = _layer2_call(p2, dinv_d, snorm, W2, b2r, g2r, be2r, h1)
    return hg


# speedup vs baseline: 7.6928x; 1.1831x over previous
import jax
import jax.numpy as jnp
from jax import lax
from jax.experimental import pallas as pl
from jax.experimental.pallas import tpu as pltpu
from jax.experimental.pallas import tpu_sc as plsc

N = 10000
D = 128
E = 320000

K = 128
NC = 2
NS = 16
NW = NC * NS
CH_AGG = 80
E_PAD = NW * CH_AGG * K
NP = 10240
RPT = NP // NS

_MESH = plsc.VectorSubcoreMesh(core_axis_name="c", subcore_axis_name="s")


def _zero_vmem_2d(ref, rows, width):
    z = jnp.zeros((16,), jnp.float32)

    def body(i, _):
        for k in range(width // 16):
            ref[i, pl.ds(16 * k, 16)] = z
        return 0

    lax.fori_loop(0, rows, body, 0)


def _fill_vmem_2d(ref, rows, width, value):
    v = jnp.full((16,), value, jnp.float32)

    def body(i, _):
        for k in range(width // 16):
            ref[i, pl.ds(16 * k, 16)] = v
        return 0

    lax.fori_loop(0, rows, body, 0)


DEG_EPS = 2.0 ** -12
DW = 16


def _deg_body(e_hbm, out_hbm, shared_deg, idx_v, val_v):
    c = lax.axis_index("c")
    s = lax.axis_index("s")
    w = c * NS + s
    r0 = s * RPT

    _zero_vmem_2d(val_v, K, DW)
    for q in range(RPT // K):
        pltpu.sync_copy(val_v, shared_deg.at[pl.ds(r0 + K * q, K)])
    plsc.subcore_barrier()

    pltpu.sync_copy(e_hbm.at[0, w], idx_v)
    _fill_vmem_2d(val_v, K, DW, 1.0)

    def body(j, _):
        pltpu.sync_copy(val_v, shared_deg.at[idx_v.at[j]], add=True)
        return 0

    lax.fori_loop(0, CH_AGG, body, 0)

    pltpu.sync_copy(e_hbm.at[1, w], idx_v)
    _fill_vmem_2d(val_v, K, DW, DEG_EPS)
    lax.fori_loop(0, CH_AGG, body, 0)
    plsc.subcore_barrier()

    pltpu.sync_copy(shared_deg.at[pl.ds(r0, RPT)], out_hbm.at[c, s])


def _deg_call(e_agg):
    return pl.kernel(
        _deg_body,
        out_type=jax.ShapeDtypeStruct((NC, NS, RPT, DW), jnp.float32),
        mesh=_MESH,
        scratch_types=[
            pltpu.VMEM_SHARED((NP, DW), jnp.float32),
            pltpu.VMEM((CH_AGG, K), jnp.int32),
            pltpu.VMEM((K, DW), jnp.float32),
        ],
    )(e_agg)


def _agg_body(x_hbm, e_hbm, out_hbm, shared_agg, idx_s, idx_d, rows_v, sem):
    c = lax.axis_index("c")
    s = lax.axis_index("s")
    w = c * NS + s
    r0 = s * RPT

    pltpu.sync_copy(e_hbm.at[0, w], idx_s)
    pltpu.sync_copy(e_hbm.at[1, w], idx_d)

    _zero_vmem_2d(rows_v, K, D)
    for q in range(RPT // K):
        pltpu.sync_copy(rows_v, shared_agg.at[pl.ds(r0 + K * q, K)])
    plsc.subcore_barrier()

    def body(j, _):
        pltpu.async_copy(x_hbm.at[idx_s.at[j]], rows_v, sem).wait()
        pltpu.sync_copy(rows_v, shared_agg.at[idx_d.at[j]], add=True)
        return 0

    lax.fori_loop(0, CH_AGG, body, 0)
    plsc.subcore_barrier()

    pltpu.sync_copy(shared_agg.at[pl.ds(r0, RPT)], out_hbm.at[c, s])


def _agg_call(x, e_agg):
    return pl.kernel(
        _agg_body,
        out_type=jax.ShapeDtypeStruct((NC, NS, RPT, D), jnp.float32),
        mesh=_MESH,
        scratch_types=[
            pltpu.VMEM_SHARED((NP, D), jnp.float32),
            pltpu.VMEM((CH_AGG, K), jnp.int32),
            pltpu.VMEM((CH_AGG, K), jnp.int32),
            pltpu.VMEM((K, D), jnp.float32),
            pltpu.SemaphoreType.DMA,
        ],
    )(x, e_agg)


def _pre_body(nf_ref, dp_ref, x1_ref, ds_ref, dd_ref):
    t = dp_ref[0] + dp_ref[1]
    dsrc = jnp.floor(t)
    ddst = (t - dsrc) * (1.0 / DEG_EPS)
    dinv_s = lax.rsqrt(jnp.maximum(dsrc, 1.0))
    dinv_d = lax.rsqrt(jnp.maximum(ddst, 1.0))
    ds_ref[...] = dinv_s
    dd_ref[...] = dinv_d
    x1_ref[pl.ds(0, N), :] = nf_ref[...] * dinv_s
    x1_ref[pl.ds(N, NP - N), :] = jnp.zeros((NP - N, D), jnp.float32)


def _pre_call(nodes_feat, deg_partials):
    return pl.pallas_call(
        _pre_body,
        out_shape=(
            jax.ShapeDtypeStruct((NP, D), jnp.float32),
            jax.ShapeDtypeStruct((N, 1), jnp.float32),
            jax.ShapeDtypeStruct((N, 1), jnp.float32),
        ),
    )(nodes_feat, deg_partials)


def _dense_common(p_ref, dd_ref, snorm_ref, w_ref, b_ref, g_ref, be_ref,
                  h_ref):
    agg = (p_ref[0] + p_ref[1]) * dd_ref[...]
    y = jnp.dot(agg, w_ref[...], preferred_element_type=jnp.float32)
    y = (y + b_ref[...]) * snorm_ref[...]
    mu = jnp.mean(y, axis=0, keepdims=True)
    yc = y - mu
    var = jnp.mean(yc * yc, axis=0, keepdims=True)
    yn = yc * lax.rsqrt(var + 1e-5) * g_ref[...] + be_ref[...]
    return h_ref[...] + jnp.maximum(yn, 0.0)


def _layer1_body(p_ref, dd_ref, snorm_ref, w_ref, b_ref, g_ref, be_ref,
                 h_ref, ds_ref, h1_ref, x2_ref):
    h1 = _dense_common(p_ref, dd_ref, snorm_ref, w_ref, b_ref, g_ref,
                       be_ref, h_ref)
    h1_ref[...] = h1
    x2_ref[pl.ds(0, N), :] = h1 * ds_ref[...]
    x2_ref[pl.ds(N, NP - N), :] = jnp.zeros((NP - N, D), jnp.float32)


def _layer1_call(p, dinv_dst, snorm, w, b, g, be, h_in, dinv_src):
    return pl.pallas_call(
        _layer1_body,
        out_shape=(
            jax.ShapeDtypeStruct((N, D), jnp.float32),
            jax.ShapeDtypeStruct((NP, D), jnp.float32),
        ),
    )(p, dinv_dst, snorm, w, b, g, be, h_in, dinv_src)


def _layer2_body(p_ref, dd_ref, snorm_ref, w_ref, b_ref, g_ref, be_ref,
                 h_ref, hg_ref):
    h2 = _dense_common(p_ref, dd_ref, snorm_ref, w_ref, b_ref, g_ref,
                       be_ref, h_ref)
    hg_ref[...] = jnp.mean(h2, axis=0, keepdims=True)


def _layer2_call(p, dinv_dst, snorm, w, b, g, be, h_in):
    return pl.pallas_call(
        _layer2_body,
        out_shape=jax.ShapeDtypeStruct((1, D), jnp.float32),
    )(p, dinv_dst, snorm, w, b, g, be, h_in)


def kernel(edge_index, nodes_feat, edges_feat, nodes_num_norm_sqrt,
           edges_num_norm_sqrt, W1, b1, gamma1, beta1, W2, b2, gamma2, beta2):
    del edges_feat, edges_num_norm_sqrt
    ei = edge_index.astype(jnp.int32)
    padv = N + (jnp.arange(E_PAD - E, dtype=jnp.int32) % (NP - N))
    e_all = jnp.concatenate([ei, jnp.broadcast_to(padv, (2, E_PAD - E))], axis=1)
    e_agg = e_all.reshape(2, NW, CH_AGG, K)

    deg_p = _deg_call(e_agg).reshape(NC, NP, DW)[:, :N, 0:1]

    b1r, g1r, be1r = b1.reshape(1, D), gamma1.reshape(1, D), beta1.reshape(1, D)
    b2r, g2r, be2r = b2.reshape(1, D), gamma2.reshape(1, D), beta2.reshape(1, D)
    snorm = nodes_num_norm_sqrt

    x1, dinv_s, dinv_d = _pre_call(nodes_feat, deg_p)
    p1 = _agg_call(x1, e_agg).reshape(2, NP, D)[:, :N]
    h1, x2 = _layer1_call(p1, dinv_d, snorm, W1, b1r, g1r, be1r,
                          nodes_feat, dinv_s)
    p2 = _agg_call(x2, e_agg).reshape(2, NP, D)[:, :N]
    hg = _layer2_call(p2, dinv_d, snorm, W2, b2r, g2r, be2r, h1)
    return hg
